# baseline (speedup 1.0000x reference)
import jax
import jax.numpy as jnp
from jax import lax
from jax.experimental import pallas as pl
from jax.experimental.pallas import tpu as pltpu
from jax.experimental.pallas import tpu_sc as plsc

NUM_CORES = 2
NUM_SUBCORES = 16
LANES = 16
NW = NUM_CORES * NUM_SUBCORES

NUM_ROWS = 1000000
BATCH = 16384
DIM = 64
B_PER_W = BATCH // NW
CHUNK = 128
N_CHUNKS = B_PER_W // CHUNK
GROUPS = CHUNK // LANES

TBLK = 512
TGRID = -(-NUM_ROWS // TBLK)
TROWS = TGRID * (TBLK // 2)


def _tpose_body(in_ref, out_ref):
    x = in_ref[...]
    eye = jnp.eye(DIM, dtype=jnp.float32)
    xt = jax.lax.dot_general(x, eye, (((0,), (0,)), ((), ())),
                             preferred_element_type=jnp.float32)
    out_ref[...] = jnp.concatenate(
        [xt[:TBLK // 2], xt[TBLK // 2:]], axis=1)


def _transpose_table(tt):
    return pl.pallas_call(
        _tpose_body,
        grid=(TGRID,),
        in_specs=[pl.BlockSpec((DIM, TBLK), lambda i: (0, i))],
        out_specs=pl.BlockSpec((TBLK // 2, 128), lambda i: (i, 0)),
        out_shape=jax.ShapeDtypeStruct((TROWS, 128), jnp.float32),
    )(tt)


def _sc_body(users_hbm, items_hbm, tu_hbm, it_hbm, out_hbm,
             uidx_v, iidx_v, urows_v, irows_v, scores_v,
             usem, isem):
    wid = lax.axis_index("s") * NUM_CORES + lax.axis_index("c")
    base_chunk = wid * N_CHUNKS

    pltpu.sync_copy(users_hbm.at[pl.ds(base_chunk, N_CHUNKS)], uidx_v)
    pltpu.sync_copy(items_hbm.at[pl.ds(base_chunk, N_CHUNKS)], iidx_v)

    def remap_body(q, _):
        cq, gq = q // GROUPS, (q % GROUPS) * LANES
        u = uidx_v[cq, pl.ds(gq, LANES)]
        m = u & (TBLK - 1)
        uidx_v[cq, pl.ds(gq, LANES)] = (
            (u - m) + 2 * (m & (TBLK // 2 - 1)) + lax.shift_right_logical(
                m, 8))
        return 0

    lax.fori_loop(0, N_CHUNKS * GROUPS, remap_body, 0, unroll=4)

    lane_iota = lax.iota(jnp.int32, LANES)

    for c in range(N_CHUNKS):
        ucp = pltpu.async_copy(tu_hbm.at[uidx_v.at[c]], urows_v, usem)
        icp = pltpu.async_copy(it_hbm.at[iidx_v.at[c]], irows_v, isem)
        ucp.wait()
        icp.wait()

        def group_body(g, _):
            base = g * LANES
            col = jnp.zeros((LANES,), jnp.float32)
            for j in range(LANES):
                prod = jnp.zeros((LANES,), jnp.float32)
                for k in range(DIM // LANES):
                    u = urows_v[base + j, pl.ds(k * LANES, LANES)]
                    v = irows_v[base + j, pl.ds(k * LANES, LANES)]
                    prod = prod + u * v
                col = jnp.where(lane_iota == j, jnp.sum(prod), col)
            score = 1.0 / (1.0 + jnp.exp(-col))
            scores_v[pl.ds(c * CHUNK + g * LANES, LANES)] = score
            return 0

        lax.fori_loop(0, GROUPS, group_body, 0)

    pltpu.sync_copy(scores_v, out_hbm.at[pl.ds(wid * B_PER_W, B_PER_W)])


@jax.jit
def kernel(users, items, user_table, item_table):
    trans_u = _transpose_table(user_table.T)
    users2 = users.reshape(BATCH // CHUNK, CHUNK)
    items2 = items.reshape(BATCH // CHUNK, CHUNK)
    mesh = plsc.VectorSubcoreMesh(core_axis_name="c", subcore_axis_name="s")
    run = pl.kernel(
        _sc_body,
        out_type=jax.ShapeDtypeStruct((BATCH,), jnp.float32),
        mesh=mesh,
        scratch_types=[
            pltpu.VMEM((N_CHUNKS, CHUNK), jnp.int32),
            pltpu.VMEM((N_CHUNKS, CHUNK), jnp.int32),
            pltpu.VMEM((CHUNK, DIM), jnp.float32),
            pltpu.VMEM((CHUNK, DIM), jnp.float32),
            pltpu.VMEM((B_PER_W,), jnp.float32),
            pltpu.SemaphoreType.DMA,
            pltpu.SemaphoreType.DMA,
        ],
        compiler_params=pltpu.CompilerParams(
            needs_layout_passes=False, use_tc_tiling_on_sc=False),
    )
    return run(users2, items2, trans_u.reshape(2 * TROWS, DIM), item_table)


# speedup vs baseline: 2.1390x; 2.1390x over previous
import jax
import jax.numpy as jnp
from jax import lax
from jax.experimental import pallas as pl
from jax.experimental.pallas import tpu as pltpu
from jax.experimental.pallas import tpu_sc as plsc

NUM_CORES = 2
NUM_SUBCORES = 16
LANES = 16
NW = NUM_CORES * NUM_SUBCORES

NUM_ROWS = 1000000
BATCH = 16384
DIM = 64
B_PER_W = BATCH // NW
CHUNK = 128
N_CHUNKS = B_PER_W // CHUNK
GROUPS = CHUNK // LANES

TBLK = 4096
TGRID = -(-NUM_ROWS // TBLK)
TROWS = TGRID * (TBLK // 2)
TSHIFT = (TBLK // 2).bit_length() - 1


def _tpose_body(in_ref, out_ref):
    x = in_ref[...]
    xt = x.T
    out_ref[...] = jnp.concatenate(
        [xt[:TBLK // 2], xt[TBLK // 2:]], axis=1)


def _transpose_table(tt):
    return pl.pallas_call(
        _tpose_body,
        grid=(TGRID,),
        in_specs=[pl.BlockSpec((DIM, TBLK), lambda i: (0, i))],
        out_specs=pl.BlockSpec((TBLK // 2, 128), lambda i: (i, 0)),
        out_shape=jax.ShapeDtypeStruct((TROWS, 128), jnp.float32),
    )(tt)


def _sc_body(users_hbm, items_hbm, tu_hbm, it_hbm, out_hbm,
             uidx_v, iidx_v, urows_v, irows_v, scores_v,
             usem, isem):
    wid = lax.axis_index("s") * NUM_CORES + lax.axis_index("c")
    base_chunk = wid * N_CHUNKS

    pltpu.sync_copy(users_hbm.at[pl.ds(base_chunk, N_CHUNKS)], uidx_v)
    pltpu.sync_copy(items_hbm.at[pl.ds(base_chunk, N_CHUNKS)], iidx_v)

    def remap_body(q, _):
        cq, gq = q // GROUPS, (q % GROUPS) * LANES
        u = uidx_v[cq, pl.ds(gq, LANES)]
        m = u & (TBLK - 1)
        uidx_v[cq, pl.ds(gq, LANES)] = (
            (u - m) + 2 * (m & (TBLK // 2 - 1)) + lax.shift_right_logical(
                m, TSHIFT))
        return 0

    lax.fori_loop(0, N_CHUNKS * GROUPS, remap_body, 0, unroll=4)

    lane_iota = lax.iota(jnp.int32, LANES)

    for c in range(N_CHUNKS):
        ucp = pltpu.async_copy(tu_hbm.at[uidx_v.at[c]], urows_v, usem)
        icp = pltpu.async_copy(it_hbm.at[iidx_v.at[c]], irows_v, isem)
        ucp.wait()
        icp.wait()

        def group_body(g, _):
            base = g * LANES
            col = jnp.zeros((LANES,), jnp.float32)
            for j in range(LANES):
                prod = jnp.zeros((LANES,), jnp.float32)
                for k in range(DIM // LANES):
                    u = urows_v[base + j, pl.ds(k * LANES, LANES)]
                    v = irows_v[base + j, pl.ds(k * LANES, LANES)]
                    prod = prod + u * v
                col = jnp.where(lane_iota == j, jnp.sum(prod), col)
            score = 1.0 / (1.0 + jnp.exp(-col))
            scores_v[pl.ds(c * CHUNK + g * LANES, LANES)] = score
            return 0

        lax.fori_loop(0, GROUPS, group_body, 0)

    pltpu.sync_copy(scores_v, out_hbm.at[pl.ds(wid * B_PER_W, B_PER_W)])


@jax.jit
def kernel(users, items, user_table, item_table):
    trans_u = _transpose_table(user_table.T)
    users2 = users.reshape(BATCH // CHUNK, CHUNK)
    items2 = items.reshape(BATCH // CHUNK, CHUNK)
    mesh = plsc.VectorSubcoreMesh(core_axis_name="c", subcore_axis_name="s")
    run = pl.kernel(
        _sc_body,
        out_type=jax.ShapeDtypeStruct((BATCH,), jnp.float32),
        mesh=mesh,
        scratch_types=[
            pltpu.VMEM((N_CHUNKS, CHUNK), jnp.int32),
            pltpu.VMEM((N_CHUNKS, CHUNK), jnp.int32),
            pltpu.VMEM((CHUNK, DIM), jnp.float32),
            pltpu.VMEM((CHUNK, DIM), jnp.float32),
            pltpu.VMEM((B_PER_W,), jnp.float32),
            pltpu.SemaphoreType.DMA,
            pltpu.SemaphoreType.DMA,
        ],
        compiler_params=pltpu.CompilerParams(
            needs_layout_passes=False, use_tc_tiling_on_sc=False),
    )
    return run(users2, items2, trans_u.reshape(2 * TROWS, DIM), item_table)


# speedup vs baseline: 2.3329x; 1.0906x over previous
import jax
import jax.numpy as jnp
from jax import lax
from jax.experimental import pallas as pl
from jax.experimental.pallas import tpu as pltpu
from jax.experimental.pallas import tpu_sc as plsc

NUM_CORES = 2
NUM_SUBCORES = 16
LANES = 16
NW = NUM_CORES * NUM_SUBCORES

NUM_ROWS = 1000000
BATCH = 16384
DIM = 64
B_PER_W = BATCH // NW
CHUNK = 128
N_CHUNKS = B_PER_W // CHUNK
GROUPS = CHUNK // LANES

TBLK = 16384
TGRID = -(-NUM_ROWS // TBLK)
TROWS = TGRID * (TBLK // 2)
TSHIFT = (TBLK // 2).bit_length() - 1


def _tpose_body(in_ref, out_ref):
    x = in_ref[...]
    xt = x.T
    out_ref[...] = jnp.concatenate(
        [xt[:TBLK // 2], xt[TBLK // 2:]], axis=1)


def _transpose_table(tt):
    return pl.pallas_call(
        _tpose_body,
        grid=(TGRID,),
        in_specs=[pl.BlockSpec((DIM, TBLK), lambda i: (0, i))],
        out_specs=pl.BlockSpec((TBLK // 2, 128), lambda i: (i, 0)),
        out_shape=jax.ShapeDtypeStruct((TROWS, 128), jnp.float32),
    )(tt)


def _sc_body(users_hbm, items_hbm, tu_hbm, it_hbm, out_hbm,
             uidx_v, iidx_v, urows_v, irows_v, scores_v,
             usem, isem):
    wid = lax.axis_index("s") * NUM_CORES + lax.axis_index("c")
    base_chunk = wid * N_CHUNKS

    pltpu.sync_copy(users_hbm.at[pl.ds(base_chunk, N_CHUNKS)], uidx_v)
    pltpu.sync_copy(items_hbm.at[pl.ds(base_chunk, N_CHUNKS)], iidx_v)

    def remap_body(q, _):
        cq, gq = q // GROUPS, (q % GROUPS) * LANES
        u = uidx_v[cq, pl.ds(gq, LANES)]
        m = u & (TBLK - 1)
        uidx_v[cq, pl.ds(gq, LANES)] = (
            (u - m) + 2 * (m & (TBLK // 2 - 1)) + lax.shift_right_logical(
                m, TSHIFT))
        return 0

    lax.fori_loop(0, N_CHUNKS * GROUPS, remap_body, 0, unroll=4)

    lane_iota = lax.iota(jnp.int32, LANES)

    for c in range(N_CHUNKS):
        ucp = pltpu.async_copy(tu_hbm.at[uidx_v.at[c]], urows_v, usem)
        icp = pltpu.async_copy(it_hbm.at[iidx_v.at[c]], irows_v, isem)
        ucp.wait()
        icp.wait()

        def group_body(g, _):
            base = g * LANES
            col = jnp.zeros((LANES,), jnp.float32)
            for j in range(LANES):
                prod = jnp.zeros((LANES,), jnp.float32)
                for k in range(DIM // LANES):
                    u = urows_v[base + j, pl.ds(k * LANES, LANES)]
                    v = irows_v[base + j, pl.ds(k * LANES, LANES)]
                    prod = prod + u * v
                col = jnp.where(lane_iota == j, jnp.sum(prod), col)
            score = 1.0 / (1.0 + jnp.exp(-col))
            scores_v[pl.ds(c * CHUNK + g * LANES, LANES)] = score
            return 0

        lax.fori_loop(0, GROUPS, group_body, 0)

    pltpu.sync_copy(scores_v, out_hbm.at[pl.ds(wid * B_PER_W, B_PER_W)])


@jax.jit
def kernel(users, items, user_table, item_table):
    trans_u = _transpose_table(user_table.T)
    users2 = users.reshape(BATCH // CHUNK, CHUNK)
    items2 = items.reshape(BATCH // CHUNK, CHUNK)
    mesh = plsc.VectorSubcoreMesh(core_axis_name="c", subcore_axis_name="s")
    run = pl.kernel(
        _sc_body,
        out_type=jax.ShapeDtypeStruct((BATCH,), jnp.float32),
        mesh=mesh,
        scratch_types=[
            pltpu.VMEM((N_CHUNKS, CHUNK), jnp.int32),
            pltpu.VMEM((N_CHUNKS, CHUNK), jnp.int32),
            pltpu.VMEM((CHUNK, DIM), jnp.float32),
            pltpu.VMEM((CHUNK, DIM), jnp.float32),
            pltpu.VMEM((B_PER_W,), jnp.float32),
            pltpu.SemaphoreType.DMA,
            pltpu.SemaphoreType.DMA,
        ],
        compiler_params=pltpu.CompilerParams(
            needs_layout_passes=False, use_tc_tiling_on_sc=False),
    )
    return run(users2, items2, trans_u.reshape(2 * TROWS, DIM), item_table)
